# Initial kernel scaffold; baseline (speedup 1.0000x reference)
import jax
import jax.numpy as jnp
from jax.experimental import pallas as pl


def kernel(x, edge_index, Wf, af_src, af_dst, bf, Wu, au_src, au_dst, bu, Wfc, bfc, Wo, ao_src, ao_dst, bo):
    raise NotImplementedError("write your pallas kernel here")


# speedup vs baseline: 59.8436x; 59.8436x over previous
import jax
import jax.numpy as jnp
from jax import lax
from jax.experimental import pallas as pl
from jax.experimental.pallas import tpu as pltpu
from jax.experimental.pallas import tpu_sc as plsc

N = 10000
E = 320000
NT = 16
EPT = E // NT
B = 80
NBLK = EPT // B
R = 400


def _mm_a(x_ref, w_ref, a_ref, h_ref, va_ref):
    xb = x_ref[...]
    h = jnp.dot(xb, w_ref[0], preferred_element_type=jnp.float32)
    h_ref[...] = h
    va_ref[...] = jnp.dot(h, a_ref[0], preferred_element_type=jnp.float32)


def _mm_b(of_ref, ou_ref, w1_ref, w2_ref, bf_ref, bu_ref, bfc_ref, wo3_ref, o_ref):
    a = jnp.maximum(of_ref[...] + bf_ref[...], 0.0)
    b = jnp.maximum(ou_ref[...] + bu_ref[...], 0.0)
    h2 = jnp.dot(a, w1_ref[...], preferred_element_type=jnp.float32)
    h2 = h2 + jnp.dot(b, w2_ref[...], preferred_element_type=jnp.float32)
    h2 = jnp.maximum(h2 + bfc_ref[...], 0.0)
    o_ref[...] = jnp.dot(h2, wo3_ref[...], preferred_element_type=jnp.float32)


def _conv_body(edge_ref, h_ref, va_ref, out_ref,
               den_sp, out_sp, va_v, den_v,
               es, ed, sidx, didx, dn0, dn1, ex0, ex1, at0, at1,
               rows, zb, sem):
    c = lax.axis_index("c")
    w = lax.axis_index("s")
    nstart = 640 * w
    ncnt = jnp.where(w < 15, 640, 400)
    nblk_node = ncnt // 80
    i16 = lax.iota(jnp.int32, 16)

    pltpu.sync_copy(va_ref.at[pl.ds(c * (4 * N), 4 * N)], va_v)

    def _z(i, _):
        zb[pl.ds(i * 16, 16)] = jnp.zeros((16,), jnp.float32)
        for k in range(4):
            rows[i, pl.ds(k * 16, 16)] = jnp.zeros((16,), jnp.float32)
        return 0
    lax.fori_loop(0, B, _z, 0)

    def _zero_out(k, _):
        pltpu.sync_copy(rows, out_sp.at[pl.ds(nstart + k * 80, 80)])
        pltpu.sync_copy(zb, den_sp.at[pl.ds(2 * nstart + k * 160, 80)])
        pltpu.sync_copy(zb, den_sp.at[pl.ds(2 * nstart + k * 160 + 80, 80)])
        return 0
    lax.fori_loop(0, nblk_node, _zero_out, 0)
    plsc.subcore_barrier()

    def _ex(s16, d16):
        a0 = plsc.load_gather(va_v, [s16 * 4]) + plsc.load_gather(va_v, [d16 * 4 + 2])
        a1 = plsc.load_gather(va_v, [s16 * 4 + 1]) + plsc.load_gather(va_v, [d16 * 4 + 3])
        a0 = jnp.where(a0 >= 0, a0, 0.2 * a0)
        a1 = jnp.where(a1 >= 0, a1, 0.2 * a1)
        return jnp.exp(a0), jnp.exp(a1)

    def _p1_sub(s16, d16, sl):
        e0, e1 = _ex(s16, d16)
        ex0[sl] = e0
        ex1[sl] = e1
        dn0[sl] = d16 * 2
        dn1[sl] = d16 * 2 + 1

    def _p1_edges(i, _):
        eoff = w * EPT + i * B
        pltpu.sync_copy(edge_ref.at[pl.ds(c * E + eoff, B)], es)
        pltpu.sync_copy(edge_ref.at[pl.ds((1 - c) * E + eoff, B)], ed)
        for j in range(5):
            sl = pl.ds(j * 16, 16)
            _p1_sub(es[sl], ed[sl], sl)
        pltpu.sync_copy(ex0, den_sp.at[dn0], add=True)
        pltpu.sync_copy(ex1, den_sp.at[dn1], add=True)
        return 0
    lax.fori_loop(0, NBLK, _p1_edges, 0)

    def _p1_loops(i, _):
        base = nstart + i * 80
        for j in range(5):
            sl = pl.ds(j * 16, 16)
            n16 = base + j * 16 + i16
            _p1_sub(n16, n16, sl)
        pltpu.sync_copy(ex0, den_sp.at[dn0], add=True)
        pltpu.sync_copy(ex1, den_sp.at[dn1], add=True)
        return 0
    lax.fori_loop(0, nblk_node, _p1_loops, 0)

    plsc.subcore_barrier()
    pltpu.sync_copy(den_sp, den_v)

    def _p2_sub(s16, d16, sl):
        e0, e1 = _ex(s16, d16)
        q0 = plsc.load_gather(den_v, [d16 * 2]) + 1e-16
        q1 = plsc.load_gather(den_v, [d16 * 2 + 1]) + 1e-16
        at0[sl] = e0 / q0
        at1[sl] = e1 / q1
        sidx[sl] = s16 + c * N
        didx[sl] = d16

    def _p2_post():
        pltpu.async_copy(h_ref.at[sidx], rows, sem).wait()
        for j in range(5):
            a0v = at0[pl.ds(j * 16, 16)]
            a1v = at1[pl.ds(j * 16, 16)]
            for e in range(16):
                r = j * 16 + e
                a0 = a0v[e]
                a1 = a1v[e]
                rows[r, pl.ds(0, 16)] = rows[r, pl.ds(0, 16)] * a0
                rows[r, pl.ds(16, 16)] = rows[r, pl.ds(16, 16)] * a0
                rows[r, pl.ds(32, 16)] = rows[r, pl.ds(32, 16)] * a1
                rows[r, pl.ds(48, 16)] = rows[r, pl.ds(48, 16)] * a1
        pltpu.sync_copy(rows, out_sp.at[didx], add=True)

    def _p2_edges(i, _):
        eoff = w * EPT + i * B
        pltpu.sync_copy(edge_ref.at[pl.ds(c * E + eoff, B)], es)
        pltpu.sync_copy(edge_ref.at[pl.ds((1 - c) * E + eoff, B)], ed)
        for j in range(5):
            sl = pl.ds(j * 16, 16)
            _p2_sub(es[sl], ed[sl], sl)
        _p2_post()
        return 0
    lax.fori_loop(0, NBLK, _p2_edges, 0)

    def _p2_loops(i, _):
        base = nstart + i * 80
        for j in range(5):
            sl = pl.ds(j * 16, 16)
            n16 = base + j * 16 + i16
            _p2_sub(n16, n16, sl)
        _p2_post()
        return 0
    lax.fori_loop(0, nblk_node, _p2_loops, 0)

    plsc.subcore_barrier()

    def _wb(k, _):
        sl = pl.ds(nstart + k * 80, 80)
        pltpu.sync_copy(out_sp.at[sl],
                        out_ref.at[pl.ds(c * N + nstart + k * 80, 80)])
        return 0
    lax.fori_loop(0, nblk_node, _wb, 0)


def _sc_out_body(edge_ref, val_ref, as_ref, ad_ref, bo_ref, out_ref,
                 den_sp, out_sp, val_v, as_v, ad_v, den_v,
                 es, ed, didx, exb, msg, zb, bo_v):
    c = lax.axis_index("c")
    w = lax.axis_index("s")
    nstart = 640 * w
    ncnt = jnp.where(w < 15, 640, 400)
    nblk_node = ncnt // 80
    i16 = lax.iota(jnp.int32, 16)

    pltpu.sync_copy(val_ref, val_v)
    pltpu.sync_copy(as_ref, as_v)
    pltpu.sync_copy(ad_ref, ad_v)
    pltpu.sync_copy(bo_ref, bo_v)

    def _z(i, _):
        zb[pl.ds(i * 16, 16)] = jnp.zeros((16,), jnp.float32)
        return 0
    lax.fori_loop(0, 5, _z, 0)

    def _zero(k, _):
        pltpu.sync_copy(zb, den_sp.at[pl.ds(nstart + k * 80, 80)])
        pltpu.sync_copy(zb, out_sp.at[pl.ds(nstart + k * 80, 80)])
        return 0
    lax.fori_loop(0, nblk_node, _zero, 0)
    plsc.subcore_barrier()

    def _exf(s16, d16):
        a = plsc.load_gather(as_v, [s16]) + plsc.load_gather(ad_v, [d16])
        a = jnp.where(a >= 0, a, 0.2 * a)
        return jnp.exp(a)

    def _p1(i, _):
        eoff = w * EPT + i * B
        pltpu.sync_copy(edge_ref.at[pl.ds(eoff, B)], es)
        pltpu.sync_copy(edge_ref.at[pl.ds(E + eoff, B)], ed)
        for j in range(5):
            sl = pl.ds(j * 16, 16)
            d16 = ed[sl]
            exb[sl] = _exf(es[sl], d16)
            didx[sl] = d16
        pltpu.sync_copy(exb, den_sp.at[didx], add=True)
        return 0
    lax.fori_loop(0, NBLK, _p1, 0)

    def _p1l(i, _):
        base = nstart + i * 80
        for j in range(5):
            sl = pl.ds(j * 16, 16)
            n16 = base + j * 16 + i16
            exb[sl] = _exf(n16, n16)
            didx[sl] = n16
        pltpu.sync_copy(exb, den_sp.at[didx], add=True)
        return 0
    lax.fori_loop(0, nblk_node, _p1l, 0)

    plsc.subcore_barrier()
    pltpu.sync_copy(den_sp, den_v)

    def _msg(s16, d16, sl):
        ex = _exf(s16, d16)
        q = plsc.load_gather(den_v, [d16]) + 1e-16
        msg[sl] = plsc.load_gather(val_v, [s16]) * ex / q
        didx[sl] = d16

    def _p2(i, _):
        eoff = w * EPT + i * B
        pltpu.sync_copy(edge_ref.at[pl.ds(eoff, B)], es)
        pltpu.sync_copy(edge_ref.at[pl.ds(E + eoff, B)], ed)
        for j in range(5):
            sl = pl.ds(j * 16, 16)
            _msg(es[sl], ed[sl], sl)
        pltpu.sync_copy(msg, out_sp.at[didx], add=True)
        return 0
    lax.fori_loop(0, NBLK, _p2, 0)

    def _p2l(i, _):
        base = nstart + i * 80
        for j in range(5):
            sl = pl.ds(j * 16, 16)
            n16 = base + j * 16 + i16
            _msg(n16, n16, sl)
        pltpu.sync_copy(msg, out_sp.at[didx], add=True)
        return 0
    lax.fori_loop(0, nblk_node, _p2l, 0)

    plsc.subcore_barrier()

    @pl.when(c == 0)
    def _fin_phase():
        bov = bo_v[...]

        def _fin(k, _):
            sl = pl.ds(nstart + k * 80, 80)
            pltpu.sync_copy(out_sp.at[sl], msg)
            for j in range(5):
                s2 = pl.ds(j * 16, 16)
                v = msg[s2] + bov
                msg[s2] = 1.0 / (1.0 + jnp.exp(-v))
            pltpu.sync_copy(msg, out_ref.at[sl])
            return 0
        lax.fori_loop(0, nblk_node, _fin, 0)


def kernel(x, edge_index, Wf, af_src, af_dst, bf, Wu, au_src, au_dst, bu,
           Wfc, bfc, Wo, ao_src, ao_dst, bo):
    f32 = jnp.float32
    Af = jnp.zeros((64, 4), f32)
    Af = Af.at[0:32, 0].set(af_src[0]).at[32:64, 1].set(af_src[1])
    Af = Af.at[0:32, 2].set(af_dst[0]).at[32:64, 3].set(af_dst[1])
    Au = jnp.zeros((64, 4), f32)
    Au = Au.at[0:32, 0].set(au_src[0]).at[32:64, 1].set(au_src[1])
    Au = Au.at[0:32, 2].set(au_dst[0]).at[32:64, 3].set(au_dst[1])
    Wst = jnp.stack([Wf, Wu])
    Ast = jnp.stack([Af, Au])

    h2n, va = pl.pallas_call(
        _mm_a,
        grid=(2, N // R),
        in_specs=[
            pl.BlockSpec((R, 128), lambda i, j: (j, 0)),
            pl.BlockSpec((1, 128, 64), lambda i, j: (i, 0, 0)),
            pl.BlockSpec((1, 64, 4), lambda i, j: (i, 0, 0)),
        ],
        out_specs=[
            pl.BlockSpec((R, 64), lambda i, j: (i * (N // R) + j, 0)),
            pl.BlockSpec((R, 4), lambda i, j: (i * (N // R) + j, 0)),
        ],
        out_shape=[
            jax.ShapeDtypeStruct((2 * N, 64), f32),
            jax.ShapeDtypeStruct((2 * N, 4), f32),
        ],
    )(x, Wst, Ast)

    edge_flat = edge_index.reshape(-1)
    va_flat = va.reshape(-1)

    mesh = plsc.VectorSubcoreMesh(core_axis_name="c", subcore_axis_name="s")
    out2 = pl.kernel(
        _conv_body,
        out_type=jax.ShapeDtypeStruct((2 * N, 64), f32),
        mesh=mesh,
        compiler_params=pltpu.CompilerParams(needs_layout_passes=False, use_tc_tiling_on_sc=False),
        scratch_types=[
            pltpu.VMEM_SHARED((2 * N,), f32),
            pltpu.VMEM_SHARED((N, 64), f32),
            pltpu.VMEM((4 * N,), f32),
            pltpu.VMEM((2 * N,), f32),
            pltpu.VMEM((B,), jnp.int32),
            pltpu.VMEM((B,), jnp.int32),
            pltpu.VMEM((B,), jnp.int32),
            pltpu.VMEM((B,), jnp.int32),
            pltpu.VMEM((B,), jnp.int32),
            pltpu.VMEM((B,), jnp.int32),
            pltpu.VMEM((B,), f32),
            pltpu.VMEM((B,), f32),
            pltpu.VMEM((B,), f32),
            pltpu.VMEM((B,), f32),
            pltpu.VMEM((B, 64), f32),
            pltpu.VMEM((B,), f32),
            pltpu.SemaphoreType.DMA,
        ],
    )(edge_flat, h2n, va_flat)

    w1 = Wfc[:64]
    w2 = Wfc[64:]
    wo = Wo[:, 0]
    Wo3 = jnp.zeros((64, 8), f32)
    Wo3 = Wo3.at[:, 0].set(wo)
    Wo3 = Wo3.at[:, 1].set(wo * ao_src[0, 0])
    Wo3 = Wo3.at[:, 2].set(wo * ao_dst[0, 0])

    misc2 = pl.pallas_call(
        _mm_b,
        grid=(N // R,),
        in_specs=[
            pl.BlockSpec((R, 64), lambda i: (i, 0)),
            pl.BlockSpec((R, 64), lambda i: (N // R + i, 0)),
            pl.BlockSpec((64, 64), lambda i: (0, 0)),
            pl.BlockSpec((64, 64), lambda i: (0, 0)),
            pl.BlockSpec((1, 64), lambda i: (0, 0)),
            pl.BlockSpec((1, 64), lambda i: (0, 0)),
            pl.BlockSpec((1, 64), lambda i: (0, 0)),
            pl.BlockSpec((64, 8), lambda i: (0, 0)),
        ],
        out_specs=pl.BlockSpec((R, 8), lambda i: (i, 0)),
        out_shape=jax.ShapeDtypeStruct((N, 8), f32),
    )(out2, out2, w1, w2, bf.reshape(1, 64), bu.reshape(1, 64),
      bfc.reshape(1, 64), Wo3)

    vals = misc2[:, 0]
    aso = misc2[:, 1]
    ado = misc2[:, 2]
    bo16 = jnp.full((16,), bo[0], f32)

    mesh2 = plsc.VectorSubcoreMesh(core_axis_name="c", subcore_axis_name="s")
    logits = pl.kernel(
        _sc_out_body,
        out_type=jax.ShapeDtypeStruct((N,), f32),
        mesh=mesh2,
        compiler_params=pltpu.CompilerParams(needs_layout_passes=False, use_tc_tiling_on_sc=False),
        scratch_types=[
            pltpu.VMEM_SHARED((N,), f32),
            pltpu.VMEM_SHARED((N,), f32),
            pltpu.VMEM((N,), f32),
            pltpu.VMEM((N,), f32),
            pltpu.VMEM((N,), f32),
            pltpu.VMEM((N,), f32),
            pltpu.VMEM((B,), jnp.int32),
            pltpu.VMEM((B,), jnp.int32),
            pltpu.VMEM((B,), jnp.int32),
            pltpu.VMEM((B,), f32),
            pltpu.VMEM((B,), f32),
            pltpu.VMEM((B,), f32),
            pltpu.VMEM((16,), f32),
        ],
    )(edge_flat, vals, aso, ado, bo16)

    return logits.reshape(N, 1)


# speedup vs baseline: 167.8978x; 2.8056x over previous
import jax
import jax.numpy as jnp
from jax import lax
from jax.experimental import pallas as pl
from jax.experimental.pallas import tpu as pltpu
from jax.experimental.pallas import tpu_sc as plsc

N = 10000
E = 320000
NT = 16
EPT = E // NT
B = 80
NBLK = EPT // B
CHK = 2000
NCHK = EPT // CHK
R = 400
F32 = jnp.float32


def _mm_a(x_ref, w_ref, a_ref, h_ref, va_ref):
    xb = x_ref[...]
    h = jnp.dot(xb, w_ref[0], preferred_element_type=F32)
    h_ref[...] = h
    va_ref[...] = jnp.dot(h, a_ref[0], preferred_element_type=F32)


def _mm_b(of_ref, ou_ref, w1_ref, w2_ref, bf_ref, bu_ref, bfc_ref, wo3_ref, o_ref):
    a = jnp.maximum(of_ref[...] + bf_ref[...], 0.0)
    b = jnp.maximum(ou_ref[...] + bu_ref[...], 0.0)
    h2 = jnp.dot(a, w1_ref[...], preferred_element_type=F32)
    h2 = h2 + jnp.dot(b, w2_ref[...], preferred_element_type=F32)
    h2 = jnp.maximum(h2 + bfc_ref[...], 0.0)
    o_ref[...] = jnp.dot(h2, wo3_ref[...], preferred_element_type=F32)


def _conv_body(edge_ref, h_ref, va_ref, out_ref,
               den_sp, out_sp, va_v, den_v, es0, ed0, es1, ed1,
               sidxA, didxA, at0A, at1A, rowsA,
               sidxB, didxB, at0B, at1B, rowsB,
               dn0, dn1, ex0, ex1, zb, semA, semB, semE0, semE1):
    c = lax.axis_index("c")
    w = lax.axis_index("s")
    nstart = 640 * w
    ncnt = jnp.where(w < 15, 640, 400)
    nblk_node = ncnt // 80
    i16 = lax.iota(jnp.int32, 16)

    pltpu.sync_copy(va_ref.at[pl.ds(c * (4 * N), 4 * N)], va_v)

    def _start_edges(ch, es, ed, sem):
        pltpu.async_copy(
            edge_ref.at[pl.ds(c * E + w * EPT + ch * CHK, CHK)], es, sem)
        pltpu.async_copy(
            edge_ref.at[pl.ds((1 - c) * E + w * EPT + ch * CHK, CHK)], ed, sem)

    def _wait_edges(ch, es, ed, sem):
        pltpu.make_async_copy(
            edge_ref.at[pl.ds(c * E + w * EPT + ch * CHK, CHK)], es, sem).wait()
        pltpu.make_async_copy(
            edge_ref.at[pl.ds((1 - c) * E + w * EPT + ch * CHK, CHK)], ed, sem).wait()

    def _chunked(process):
        pltpu.sync_copy(edge_ref.at[pl.ds(c * E + w * EPT, CHK)], es0)
        pltpu.sync_copy(edge_ref.at[pl.ds((1 - c) * E + w * EPT, CHK)], ed0)

        def _pair(kk, _):
            _start_edges(2 * kk + 1, es1, ed1, semE1)
            process(es0, ed0)
            _wait_edges(2 * kk + 1, es1, ed1, semE1)

            @pl.when(kk < NCHK // 2 - 1)
            def _pf():
                _start_edges(2 * kk + 2, es0, ed0, semE0)
            process(es1, ed1)

            @pl.when(kk < NCHK // 2 - 1)
            def _pfw():
                _wait_edges(2 * kk + 2, es0, ed0, semE0)
            return 0
        lax.fori_loop(0, NCHK // 2, _pair, 0)

    def _z(i, _):
        zb[pl.ds(i * 16, 16)] = jnp.zeros((16,), F32)
        for k in range(4):
            rowsA[i, pl.ds(k * 16, 16)] = jnp.zeros((16,), F32)
        return 0
    lax.fori_loop(0, B, _z, 0)

    def _zero_out(k, _):
        pltpu.sync_copy(rowsA, out_sp.at[pl.ds(nstart + k * 80, 80)])
        pltpu.sync_copy(zb, den_sp.at[pl.ds(2 * nstart + k * 160, 80)])
        pltpu.sync_copy(zb, den_sp.at[pl.ds(2 * nstart + k * 160 + 80, 80)])
        return 0
    lax.fori_loop(0, nblk_node, _zero_out, 0)
    plsc.subcore_barrier()

    def _ex(s16, d16):
        a0 = plsc.load_gather(va_v, [s16 * 4]) + plsc.load_gather(va_v, [d16 * 4 + 2])
        a1 = plsc.load_gather(va_v, [s16 * 4 + 1]) + plsc.load_gather(va_v, [d16 * 4 + 3])
        a0 = jnp.where(a0 >= 0, a0, 0.2 * a0)
        a1 = jnp.where(a1 >= 0, a1, 0.2 * a1)
        return jnp.exp(a0), jnp.exp(a1)

    def _p1_sub(s16, d16, sl):
        e0, e1 = _ex(s16, d16)
        ex0[sl] = e0
        ex1[sl] = e1
        dn0[sl] = d16 * 2
        dn1[sl] = d16 * 2 + 1

    def _p1_chunk(es, ed):
        def _blk(bi, _):
            for j in range(5):
                sl = pl.ds(j * 16, 16)
                eo = pl.ds(bi * B + j * 16, 16)
                _p1_sub(es[eo], ed[eo], sl)
            pltpu.sync_copy(ex0, den_sp.at[dn0], add=True)
            pltpu.sync_copy(ex1, den_sp.at[dn1], add=True)
            return 0
        lax.fori_loop(0, CHK // B, _blk, 0)
    _chunked(_p1_chunk)

    def _p1_loops(i, _):
        base = nstart + i * 80
        for j in range(5):
            sl = pl.ds(j * 16, 16)
            n16 = base + j * 16 + i16
            _p1_sub(n16, n16, sl)
        pltpu.sync_copy(ex0, den_sp.at[dn0], add=True)
        pltpu.sync_copy(ex1, den_sp.at[dn1], add=True)
        return 0
    lax.fori_loop(0, nblk_node, _p1_loops, 0)

    plsc.subcore_barrier()
    pltpu.sync_copy(den_sp, den_v)

    def _fill(i, es, ed, sidx, didx, at0, at1):
        for j in range(5):
            sl = pl.ds(j * 16, 16)
            eo = pl.ds(i * B + j * 16, 16)
            s16 = es[eo]
            d16 = ed[eo]
            e0, e1 = _ex(s16, d16)
            q0 = plsc.load_gather(den_v, [d16 * 2]) + 1e-16
            q1 = plsc.load_gather(den_v, [d16 * 2 + 1]) + 1e-16
            at0[sl] = e0 / q0
            at1[sl] = e1 / q1
            sidx[sl] = s16 + c * N
            didx[sl] = d16

    def _fill_loop(i, sidx, didx, at0, at1):
        base = nstart + i * 80
        for j in range(5):
            sl = pl.ds(j * 16, 16)
            n16 = base + j * 16 + i16
            e0, e1 = _ex(n16, n16)
            q0 = plsc.load_gather(den_v, [n16 * 2]) + 1e-16
            q1 = plsc.load_gather(den_v, [n16 * 2 + 1]) + 1e-16
            at0[sl] = e0 / q0
            at1[sl] = e1 / q1
            sidx[sl] = n16 + c * N
            didx[sl] = n16

    def _startg(sidx, rows, sem):
        pltpu.async_copy(h_ref.at[sidx], rows, sem)

    def _waitg(sidx, rows, sem):
        pltpu.make_async_copy(h_ref.at[sidx], rows, sem).wait()

    def _scale_scatter(didx, at0, at1, rows):
        for j in range(5):
            a0v = at0[pl.ds(j * 16, 16)]
            a1v = at1[pl.ds(j * 16, 16)]
            for e in range(16):
                r = j * 16 + e
                a0 = a0v[e]
                a1 = a1v[e]
                rows[r, pl.ds(0, 16)] = rows[r, pl.ds(0, 16)] * a0
                rows[r, pl.ds(16, 16)] = rows[r, pl.ds(16, 16)] * a0
                rows[r, pl.ds(32, 16)] = rows[r, pl.ds(32, 16)] * a1
                rows[r, pl.ds(48, 16)] = rows[r, pl.ds(48, 16)] * a1
        pltpu.sync_copy(rows, out_sp.at[didx], add=True)

    def _p2_chunk(es, ed):
        _fill(0, es, ed, sidxA, didxA, at0A, at1A)
        _startg(sidxA, rowsA, semA)

        def _pair2(k2, _):
            _fill(2 * k2 + 1, es, ed, sidxB, didxB, at0B, at1B)
            _startg(sidxB, rowsB, semB)
            _waitg(sidxA, rowsA, semA)
            _scale_scatter(didxA, at0A, at1A, rowsA)
            _fill(2 * k2 + 2, es, ed, sidxA, didxA, at0A, at1A)
            _startg(sidxA, rowsA, semA)
            _waitg(sidxB, rowsB, semB)
            _scale_scatter(didxB, at0B, at1B, rowsB)
            return 0
        lax.fori_loop(0, (CHK // B) // 2, _pair2, 0)
        _waitg(sidxA, rowsA, semA)
        _scale_scatter(didxA, at0A, at1A, rowsA)
    _chunked(_p2_chunk)

    def _p2_loops(i, _):
        _fill_loop(i, sidxA, didxA, at0A, at1A)
        _startg(sidxA, rowsA, semA)
        _waitg(sidxA, rowsA, semA)
        _scale_scatter(didxA, at0A, at1A, rowsA)
        return 0
    lax.fori_loop(0, nblk_node, _p2_loops, 0)

    plsc.subcore_barrier()

    def _wb(k, _):
        sl = pl.ds(nstart + k * 80, 80)
        pltpu.sync_copy(out_sp.at[sl],
                        out_ref.at[pl.ds(c * N + nstart + k * 80, 80)])
        return 0
    lax.fori_loop(0, nblk_node, _wb, 0)


def _sc_out_body(edge_ref, val_ref, as_ref, ad_ref, bo_ref, out_ref,
                 den_sp, out_sp, val_v, as_v, ad_v, den_v, esb, edb,
                 didx, exb, msg, zb, bo_v):
    c = lax.axis_index("c")
    w = lax.axis_index("s")
    nstart = 640 * w
    ncnt = jnp.where(w < 15, 640, 400)
    nblk_node = ncnt // 80
    i16 = lax.iota(jnp.int32, 16)

    pltpu.sync_copy(val_ref, val_v)
    pltpu.sync_copy(as_ref, as_v)
    pltpu.sync_copy(ad_ref, ad_v)
    pltpu.sync_copy(bo_ref, bo_v)
    pltpu.sync_copy(edge_ref.at[pl.ds(w * EPT, EPT)], esb)
    pltpu.sync_copy(edge_ref.at[pl.ds(E + w * EPT, EPT)], edb)

    def _z(i, _):
        zb[pl.ds(i * 16, 16)] = jnp.zeros((16,), F32)
        return 0
    lax.fori_loop(0, 5, _z, 0)

    def _zero(k, _):
        pltpu.sync_copy(zb, den_sp.at[pl.ds(nstart + k * 80, 80)])
        pltpu.sync_copy(zb, out_sp.at[pl.ds(nstart + k * 80, 80)])
        return 0
    lax.fori_loop(0, nblk_node, _zero, 0)
    plsc.subcore_barrier()

    def _exf(s16, d16):
        a = plsc.load_gather(as_v, [s16]) + plsc.load_gather(ad_v, [d16])
        a = jnp.where(a >= 0, a, 0.2 * a)
        return jnp.exp(a)

    def _p1(i, _):
        for j in range(5):
            sl = pl.ds(j * 16, 16)
            eo = pl.ds(i * B + j * 16, 16)
            d16 = edb[eo]
            exb[sl] = _exf(esb[eo], d16)
            didx[sl] = d16
        pltpu.sync_copy(exb, den_sp.at[didx], add=True)
        return 0
    lax.fori_loop(0, NBLK, _p1, 0)

    def _p1l(i, _):
        base = nstart + i * 80
        for j in range(5):
            sl = pl.ds(j * 16, 16)
            n16 = base + j * 16 + i16
            exb[sl] = _exf(n16, n16)
            didx[sl] = n16
        pltpu.sync_copy(exb, den_sp.at[didx], add=True)
        return 0
    lax.fori_loop(0, nblk_node, _p1l, 0)

    plsc.subcore_barrier()
    pltpu.sync_copy(den_sp, den_v)

    def _msg(s16, d16, sl):
        ex = _exf(s16, d16)
        q = plsc.load_gather(den_v, [d16]) + 1e-16
        msg[sl] = plsc.load_gather(val_v, [s16]) * ex / q
        didx[sl] = d16

    def _p2(i, _):
        for j in range(5):
            sl = pl.ds(j * 16, 16)
            eo = pl.ds(i * B + j * 16, 16)
            _msg(esb[eo], edb[eo], sl)
        pltpu.sync_copy(msg, out_sp.at[didx], add=True)
        return 0
    lax.fori_loop(0, NBLK, _p2, 0)

    def _p2l(i, _):
        base = nstart + i * 80
        for j in range(5):
            sl = pl.ds(j * 16, 16)
            n16 = base + j * 16 + i16
            _msg(n16, n16, sl)
        pltpu.sync_copy(msg, out_sp.at[didx], add=True)
        return 0
    lax.fori_loop(0, nblk_node, _p2l, 0)

    plsc.subcore_barrier()

    @pl.when(c == 0)
    def _fin_phase():
        bov = bo_v[...]

        def _fin(k, _):
            sl = pl.ds(nstart + k * 80, 80)
            pltpu.sync_copy(out_sp.at[sl], msg)
            for j in range(5):
                s2 = pl.ds(j * 16, 16)
                v = msg[s2] + bov
                msg[s2] = 1.0 / (1.0 + jnp.exp(-v))
            pltpu.sync_copy(msg, out_ref.at[sl])
            return 0
        lax.fori_loop(0, nblk_node, _fin, 0)


def kernel(x, edge_index, Wf, af_src, af_dst, bf, Wu, au_src, au_dst, bu,
           Wfc, bfc, Wo, ao_src, ao_dst, bo):
    Af = jnp.zeros((64, 4), F32)
    Af = Af.at[0:32, 0].set(af_src[0]).at[32:64, 1].set(af_src[1])
    Af = Af.at[0:32, 2].set(af_dst[0]).at[32:64, 3].set(af_dst[1])
    Au = jnp.zeros((64, 4), F32)
    Au = Au.at[0:32, 0].set(au_src[0]).at[32:64, 1].set(au_src[1])
    Au = Au.at[0:32, 2].set(au_dst[0]).at[32:64, 3].set(au_dst[1])
    Wst = jnp.stack([Wf, Wu])
    Ast = jnp.stack([Af, Au])

    h2n, va = pl.pallas_call(
        _mm_a,
        grid=(2, N // R),
        in_specs=[
            pl.BlockSpec((R, 128), lambda i, j: (j, 0)),
            pl.BlockSpec((1, 128, 64), lambda i, j: (i, 0, 0)),
            pl.BlockSpec((1, 64, 4), lambda i, j: (i, 0, 0)),
        ],
        out_specs=[
            pl.BlockSpec((R, 64), lambda i, j: (i * (N // R) + j, 0)),
            pl.BlockSpec((R, 4), lambda i, j: (i * (N // R) + j, 0)),
        ],
        out_shape=[
            jax.ShapeDtypeStruct((2 * N, 64), F32),
            jax.ShapeDtypeStruct((2 * N, 4), F32),
        ],
    )(x, Wst, Ast)

    edge_flat = edge_index.reshape(-1)
    va_flat = va.reshape(-1)

    mesh = plsc.VectorSubcoreMesh(core_axis_name="c", subcore_axis_name="s")
    out2 = pl.kernel(
        _conv_body,
        out_type=jax.ShapeDtypeStruct((2 * N, 64), F32),
        mesh=mesh,
        compiler_params=pltpu.CompilerParams(
            needs_layout_passes=False, use_tc_tiling_on_sc=False),
        scratch_types=[
            pltpu.VMEM_SHARED((2 * N,), F32),
            pltpu.VMEM_SHARED((N, 64), F32),
            pltpu.VMEM((4 * N,), F32),
            pltpu.VMEM((2 * N,), F32),
            pltpu.VMEM((CHK,), jnp.int32),
            pltpu.VMEM((CHK,), jnp.int32),
            pltpu.VMEM((CHK,), jnp.int32),
            pltpu.VMEM((CHK,), jnp.int32),
            pltpu.VMEM((B,), jnp.int32),
            pltpu.VMEM((B,), jnp.int32),
            pltpu.VMEM((B,), F32),
            pltpu.VMEM((B,), F32),
            pltpu.VMEM((B, 64), F32),
            pltpu.VMEM((B,), jnp.int32),
            pltpu.VMEM((B,), jnp.int32),
            pltpu.VMEM((B,), F32),
            pltpu.VMEM((B,), F32),
            pltpu.VMEM((B, 64), F32),
            pltpu.VMEM((B,), jnp.int32),
            pltpu.VMEM((B,), jnp.int32),
            pltpu.VMEM((B,), F32),
            pltpu.VMEM((B,), F32),
            pltpu.VMEM((B,), F32),
            pltpu.SemaphoreType.DMA,
            pltpu.SemaphoreType.DMA,
            pltpu.SemaphoreType.DMA,
            pltpu.SemaphoreType.DMA,
        ],
    )(edge_flat, h2n, va_flat)

    w1 = Wfc[:64]
    w2 = Wfc[64:]
    wo = Wo[:, 0]
    Wo3 = jnp.zeros((64, 8), F32)
    Wo3 = Wo3.at[:, 0].set(wo)
    Wo3 = Wo3.at[:, 1].set(wo * ao_src[0, 0])
    Wo3 = Wo3.at[:, 2].set(wo * ao_dst[0, 0])

    misc2 = pl.pallas_call(
        _mm_b,
        grid=(N // R,),
        in_specs=[
            pl.BlockSpec((R, 64), lambda i: (i, 0)),
            pl.BlockSpec((R, 64), lambda i: (N // R + i, 0)),
            pl.BlockSpec((64, 64), lambda i: (0, 0)),
            pl.BlockSpec((64, 64), lambda i: (0, 0)),
            pl.BlockSpec((1, 64), lambda i: (0, 0)),
            pl.BlockSpec((1, 64), lambda i: (0, 0)),
            pl.BlockSpec((1, 64), lambda i: (0, 0)),
            pl.BlockSpec((64, 8), lambda i: (0, 0)),
        ],
        out_specs=pl.BlockSpec((R, 8), lambda i: (i, 0)),
        out_shape=jax.ShapeDtypeStruct((N, 8), F32),
    )(out2, out2, w1, w2, bf.reshape(1, 64), bu.reshape(1, 64),
      bfc.reshape(1, 64), Wo3)

    vals = misc2[:, 0]
    aso = misc2[:, 1]
    ado = misc2[:, 2]
    bo16 = jnp.full((16,), bo[0], F32)

    mesh2 = plsc.VectorSubcoreMesh(core_axis_name="c", subcore_axis_name="s")
    logits = pl.kernel(
        _sc_out_body,
        out_type=jax.ShapeDtypeStruct((N,), F32),
        mesh=mesh2,
        compiler_params=pltpu.CompilerParams(
            needs_layout_passes=False, use_tc_tiling_on_sc=False),
        scratch_types=[
            pltpu.VMEM_SHARED((N,), F32),
            pltpu.VMEM_SHARED((N,), F32),
            pltpu.VMEM((N,), F32),
            pltpu.VMEM((N,), F32),
            pltpu.VMEM((N,), F32),
            pltpu.VMEM((N,), F32),
            pltpu.VMEM((EPT,), jnp.int32),
            pltpu.VMEM((EPT,), jnp.int32),
            pltpu.VMEM((B,), jnp.int32),
            pltpu.VMEM((B,), F32),
            pltpu.VMEM((B,), F32),
            pltpu.VMEM((B,), F32),
            pltpu.VMEM((16,), F32),
        ],
    )(edge_flat, vals, aso, ado, bo16)

    return logits.reshape(N, 1)
